# Initial kernel scaffold; baseline (speedup 1.0000x reference)
import functools

import jax
import jax.numpy as jnp
from jax import lax
from jax.experimental import pallas as pl
from jax.experimental.pallas import tpu as pltpu
from jax.experimental.pallas import tpu_sc as plsc

F32 = jnp.float32
I32 = jnp.int32

N = 10000
D = 128
H = 128
NP = 10112
NACC = 10240
DEAD = NP
NC, NS = 2, 16
NW = NC * NS
BLK = 128
OUT_STRIPE = 632
Z_STRIPE = 640


def _cdiv(a, b):
    return (a + b - 1) // b


def _tc_pre_body(x_ref, wg_ref, asrc_ref, adst_ref, h_ref, as_ref, ad_ref,
                 es_ref):
    h = jnp.dot(x_ref[...], wg_ref[...], preferred_element_type=F32)
    h_ref[...] = h
    a_s = jnp.dot(h, asrc_ref[...], preferred_element_type=F32)
    a_d = jnp.dot(h, adst_ref[...], preferred_element_type=F32)
    as_ref[...] = a_s
    ad_ref[...] = a_d
    e = a_s + a_d
    e = jnp.where(e > 0.0, e, 0.2 * e)
    es_ref[...] = jnp.exp(e)


def _tc_pre(xp, W_gat, asrc, adst):
    br = 1264
    grid = NP // br
    return pl.pallas_call(
        _tc_pre_body,
        grid=(grid,),
        in_specs=[
            pl.BlockSpec((br, D), lambda i: (i, 0)),
            pl.BlockSpec((D, H), lambda i: (0, 0)),
            pl.BlockSpec((H, 1), lambda i: (0, 0)),
            pl.BlockSpec((H, 1), lambda i: (0, 0)),
        ],
        out_specs=[
            pl.BlockSpec((br, H), lambda i: (i, 0)),
            pl.BlockSpec((br, 1), lambda i: (i, 0)),
            pl.BlockSpec((br, 1), lambda i: (i, 0)),
            pl.BlockSpec((br, 1), lambda i: (i, 0)),
        ],
        out_shape=[
            jax.ShapeDtypeStruct((NP, H), F32),
            jax.ShapeDtypeStruct((NP, 1), F32),
            jax.ShapeDtypeStruct((NP, 1), F32),
            jax.ShapeDtypeStruct((NP, 1), F32),
        ],
    )(xp, W_gat, asrc, adst)


def _tc_mlp_body(nump_ref, denp_ref, h_ref, es_ref, bgat_ref, w1_ref, b1_ref,
                 w2_ref, v_ref):
    es = es_ref[...]
    num = nump_ref[0] + nump_ref[1] + es * h_ref[...]
    den = denp_ref[0] + denp_ref[1] + es + 1e-16
    gat = num / den + bgat_ref[...]
    hp = jnp.maximum(gat, 0.0)
    mlp = jnp.dot(hp, w1_ref[...], preferred_element_type=F32) + b1_ref[...]
    mlp = jnp.maximum(mlp, 0.0)
    v_ref[...] = jnp.dot(mlp, w2_ref[...], preferred_element_type=F32)


def _tc_mlp(nump, denp, h, es, bgat, W1, b1, W2cols):
    br = 1264
    grid = NP // br
    return pl.pallas_call(
        _tc_mlp_body,
        grid=(grid,),
        in_specs=[
            pl.BlockSpec((2, br, H), lambda i: (0, i, 0)),
            pl.BlockSpec((2, br, 1), lambda i: (0, i, 0)),
            pl.BlockSpec((br, H), lambda i: (i, 0)),
            pl.BlockSpec((br, 1), lambda i: (i, 0)),
            pl.BlockSpec((1, H), lambda i: (0, 0)),
            pl.BlockSpec((H, H), lambda i: (0, 0)),
            pl.BlockSpec((1, H), lambda i: (0, 0)),
            pl.BlockSpec((H, 8), lambda i: (0, 0)),
        ],
        out_specs=pl.BlockSpec((br, 8), lambda i: (i, 0)),
        out_shape=jax.ShapeDtypeStruct((NP, 8), F32),
    )(nump, denp, h, es, bgat, W1, b1, W2cols)


def _tc_comb1_body(agg_ref, v_ref, deg_ref, w3_ref):
    agg = agg_ref[0] + agg_ref[1]
    v = v_ref[...]
    y = 0.5 * (deg_ref[...] * v + agg)
    row = lax.broadcasted_iota(I32, y.shape, 0)
    w3_ref[...] = jnp.where(row == 0, v - y, y)


def _tc_comb1(aggA, v3, deg):
    return pl.pallas_call(
        _tc_comb1_body,
        grid=(1,),
        in_specs=[
            pl.BlockSpec((2, 3, NP), lambda i: (0, 0, 0)),
            pl.BlockSpec((3, NP), lambda i: (0, 0)),
            pl.BlockSpec((1, NP), lambda i: (0, 0)),
        ],
        out_specs=pl.BlockSpec((3, NP), lambda i: (0, 0)),
        out_shape=jax.ShapeDtypeStruct((3, NP), F32),
    )(aggA, v3, deg)


def _tc_comb2_body(agg_ref, w3_ref, deg_ref, b2_ref, out_ref):
    agg = agg_ref[0] + agg_ref[1]
    w3 = w3_ref[...]
    z = 0.5 * (deg_ref[...] * w3 + agg)
    row = lax.broadcasted_iota(I32, w3.shape, 0)
    cw = jnp.where(row == 2, 0.0, 1.0)
    cz = jnp.where(row == 2, 1.0, -1.0)
    acc = jnp.sum(cw * w3 + cz * z, axis=0, keepdims=True)
    out_ref[...] = 0.5 * acc + b2_ref[...]


def _tc_comb2(aggB, w3, deg, b2):
    return pl.pallas_call(
        _tc_comb2_body,
        grid=(1,),
        in_specs=[
            pl.BlockSpec((2, 3, NP), lambda i: (0, 0, 0)),
            pl.BlockSpec((3, NP), lambda i: (0, 0)),
            pl.BlockSpec((1, NP), lambda i: (0, 0)),
            pl.BlockSpec((1, 1), lambda i: (0, 0)),
        ],
        out_specs=pl.BlockSpec((1, NP), lambda i: (0, 0)),
        out_shape=jax.ShapeDtypeStruct((1, NP), F32),
    )(aggB, w3, deg, b2)


_MESH = plsc.VectorSubcoreMesh(core_axis_name="c", subcore_axis_name="s")
_Z16F = jnp.zeros((16,), F32)


def _gat_edges_body(nblk, src_hbm, dst_hbm, as_hbm, ad_hbm, h_hbm,
                    num_out, den_out, asb, adb, srcb, dstb, eeb, rows, sem,
                    num_sh, den_sh):
    c = lax.axis_index("c")
    s = lax.axis_index("s")
    wid = s * NC + c

    pltpu.sync_copy(as_hbm, asb)
    pltpu.sync_copy(ad_hbm, adb)

    def _zrow(i, _):
        for g in range(8):
            rows[i, pl.ds(g * 16, 16)] = _Z16F
        return 0
    lax.fori_loop(0, BLK, _zrow, 0)
    for g in range(8):
        eeb[pl.ds(g * 16, 16)] = _Z16F
    for k in range(Z_STRIPE // BLK):
        pltpu.sync_copy(rows, num_sh.at[pl.ds(s * Z_STRIPE + k * BLK, BLK)])
        pltpu.sync_copy(eeb, den_sh.at[pl.ds(s * Z_STRIPE + k * BLK, BLK)])
    plsc.subcore_barrier()

    base_e = wid * (nblk * BLK)

    def _block(b, _):
        off = base_e + b * BLK
        pltpu.sync_copy(src_hbm.at[pl.ds(off, BLK)], srcb)
        pltpu.sync_copy(dst_hbm.at[pl.ds(off, BLK)], dstb)
        for g in range(BLK // 16):
            s16 = srcb[pl.ds(g * 16, 16)]
            d16 = dstb[pl.ds(g * 16, 16)]
            e = plsc.load_gather(asb, [s16]) + plsc.load_gather(adb, [d16])
            e = jnp.where(e > 0.0, e, 0.2 * e)
            eeb[pl.ds(g * 16, 16)] = jnp.exp(e)
        pltpu.async_copy(h_hbm.at[srcb], rows, sem).wait()

        def _scale(r, _):
            spl = plsc.load_gather(eeb, [jnp.full((16,), r, I32)])
            for g in range(8):
                rows[r, pl.ds(g * 16, 16)] = rows[r, pl.ds(g * 16, 16)] * spl
            return 0
        lax.fori_loop(0, BLK, _scale, 0)

        pltpu.sync_copy(rows, num_sh.at[dstb], add=True)
        pltpu.sync_copy(eeb, den_sh.at[dstb], add=True)
        return 0
    lax.fori_loop(0, nblk, _block, 0)
    plsc.subcore_barrier()

    row0 = s * OUT_STRIPE
    pltpu.sync_copy(num_sh.at[pl.ds(row0, OUT_STRIPE)],
                    num_out.at[pl.ds(c * NP + row0, OUT_STRIPE)])
    pltpu.sync_copy(den_sh.at[pl.ds(row0, OUT_STRIPE)],
                    den_out.at[pl.ds(c * NP + row0, OUT_STRIPE)])


def _gat_edges(src_p, dst_p, a_s, a_d, h, nblk):
    fn = pl.kernel(
        functools.partial(_gat_edges_body, nblk),
        out_type=[
            jax.ShapeDtypeStruct((NC * NP, H), F32),
            jax.ShapeDtypeStruct((NC * NP,), F32),
        ],
        mesh=_MESH,
        scratch_types=[
            pltpu.VMEM((NP,), F32),
            pltpu.VMEM((NP,), F32),
            pltpu.VMEM((BLK,), I32),
            pltpu.VMEM((BLK,), I32),
            pltpu.VMEM((BLK,), F32),
            pltpu.VMEM((BLK, H), F32),
            pltpu.SemaphoreType.DMA,
            pltpu.VMEM_SHARED((NACC, H), F32),
            pltpu.VMEM_SHARED((NACC,), F32),
        ],
    )
    return fn(src_p, dst_p, a_s, a_d, h)


def _spmv3_body(nblk, src_hbm, dst_hbm, lap_hbm, x_hbm, agg_out,
                x0b, x1b, x2b, srcb, dstb, lapb, v0b, v1b, v2b,
                acc0, acc1, acc2):
    c = lax.axis_index("c")
    s = lax.axis_index("s")
    wid = s * NC + c

    pltpu.sync_copy(x_hbm.at[0], x0b)
    pltpu.sync_copy(x_hbm.at[1], x1b)
    pltpu.sync_copy(x_hbm.at[2], x2b)

    for g in range(8):
        v0b[pl.ds(g * 16, 16)] = _Z16F
    for k in range(Z_STRIPE // BLK):
        sl = pl.ds(s * Z_STRIPE + k * BLK, BLK)
        pltpu.sync_copy(v0b, acc0.at[sl])
        pltpu.sync_copy(v0b, acc1.at[sl])
        pltpu.sync_copy(v0b, acc2.at[sl])
    plsc.subcore_barrier()

    base_e = wid * (nblk * BLK)

    def _block(b, _):
        off = base_e + b * BLK
        pltpu.sync_copy(src_hbm.at[pl.ds(off, BLK)], srcb)
        pltpu.sync_copy(dst_hbm.at[pl.ds(off, BLK)], dstb)
        pltpu.sync_copy(lap_hbm.at[pl.ds(off, BLK)], lapb)
        for g in range(BLK // 16):
            sl = pl.ds(g * 16, 16)
            s16 = srcb[sl]
            lp = lapb[sl]
            v0b[sl] = lp * plsc.load_gather(x0b, [s16])
            v1b[sl] = lp * plsc.load_gather(x1b, [s16])
            v2b[sl] = lp * plsc.load_gather(x2b, [s16])
        pltpu.sync_copy(v0b, acc0.at[dstb], add=True)
        pltpu.sync_copy(v1b, acc1.at[dstb], add=True)
        pltpu.sync_copy(v2b, acc2.at[dstb], add=True)
        return 0
    lax.fori_loop(0, nblk, _block, 0)
    plsc.subcore_barrier()

    row0 = s * OUT_STRIPE
    for k, acc in enumerate((acc0, acc1, acc2)):
        pltpu.sync_copy(
            acc.at[pl.ds(row0, OUT_STRIPE)],
            agg_out.at[pl.ds((c * 3 + k) * NP + row0, OUT_STRIPE)])


def _spmv3(src_p, dst_p, lap_p, x3, nblk):
    fn = pl.kernel(
        functools.partial(_spmv3_body, nblk),
        out_type=jax.ShapeDtypeStruct((NC * 3 * NP,), F32),
        mesh=_MESH,
        scratch_types=[
            pltpu.VMEM((NP,), F32),
            pltpu.VMEM((NP,), F32),
            pltpu.VMEM((NP,), F32),
            pltpu.VMEM((BLK,), I32),
            pltpu.VMEM((BLK,), I32),
            pltpu.VMEM((BLK,), F32),
            pltpu.VMEM((BLK,), F32),
            pltpu.VMEM((BLK,), F32),
            pltpu.VMEM((BLK,), F32),
            pltpu.VMEM_SHARED((NACC,), F32),
            pltpu.VMEM_SHARED((NACC,), F32),
            pltpu.VMEM_SHARED((NACC,), F32),
        ],
    )
    return fn(src_p, dst_p, lap_p, x3)


def kernel(x, edge_index, lap_values, deg_values, W_gat, att_src, att_dst,
           b_gat, W1, b1, W2, b2):
    E = edge_index.shape[1]
    nblk = _cdiv(E, NW * BLK)
    e_pad = NW * nblk * BLK

    src = edge_index[0]
    dst = edge_index[1]
    pad = e_pad - E
    src_p = jnp.concatenate([src, jnp.zeros((pad,), I32)])
    dst_p = jnp.concatenate([dst, jnp.full((pad,), DEAD, I32)])
    lap_p = jnp.concatenate([lap_values, jnp.zeros((pad,), F32)])

    xp = jnp.pad(x, ((0, NP - N), (0, 0)))
    degp = jnp.pad(deg_values, (0, NP - N)).reshape(1, NP)

    h, a_s, a_d, es = _tc_pre(xp, W_gat, att_src.reshape(H, 1),
                              att_dst.reshape(H, 1))

    nump, denp = _gat_edges(src_p, dst_p, a_s.reshape(NP), a_d.reshape(NP),
                            h, nblk)

    w2c = jnp.concatenate(
        [W2[:, 0].reshape(3, H).T, jnp.zeros((H, 5), F32)], axis=1)
    v8 = _tc_mlp(nump.reshape(NC, NP, H), denp.reshape(NC, NP, 1), h, es,
                 b_gat.reshape(1, H), W1, b1.reshape(1, H), w2c)
    v3 = v8[:, :3].T

    aggA = _spmv3(src_p, dst_p, lap_p, v3, nblk).reshape(NC, 3, NP)
    w3 = _tc_comb1(aggA, v3, degp)
    aggB = _spmv3(src_p, dst_p, lap_p, w3, nblk).reshape(NC, 3, NP)
    out = _tc_comb2(aggB, w3, degp, b2.reshape(1, 1))
    return out[0, :N]


# speedup vs baseline: 17.2492x; 17.2492x over previous
import functools

import jax
import jax.numpy as jnp
from jax import lax
from jax.experimental import pallas as pl
from jax.experimental.pallas import tpu as pltpu
from jax.experimental.pallas import tpu_sc as plsc

F32 = jnp.float32
I32 = jnp.int32

N = 10000
D = 128
H = 128
NP = 10112
NACC = 10240
DEAD = NP
NC, NS = 2, 16
NW = NC * NS
BLK = 128
OUT_STRIPE = 632
Z_STRIPE = 640


def _cdiv(a, b):
    return (a + b - 1) // b


def _tc_pre_body(x_ref, wg_ref, asrc_ref, adst_ref, h_ref, as_ref, ad_ref,
                 es_ref):
    h = jnp.dot(x_ref[...], wg_ref[...], preferred_element_type=F32)
    h_ref[...] = h
    a_s = jnp.dot(h, asrc_ref[...], preferred_element_type=F32)
    a_d = jnp.dot(h, adst_ref[...], preferred_element_type=F32)
    as_ref[...] = a_s
    ad_ref[...] = a_d
    e = a_s + a_d
    e = jnp.where(e > 0.0, e, 0.2 * e)
    es_ref[...] = jnp.exp(e)


def _tc_pre(xp, W_gat, asrc, adst):
    br = 1264
    grid = NP // br
    return pl.pallas_call(
        _tc_pre_body,
        grid=(grid,),
        in_specs=[
            pl.BlockSpec((br, D), lambda i: (i, 0)),
            pl.BlockSpec((D, H), lambda i: (0, 0)),
            pl.BlockSpec((H, 1), lambda i: (0, 0)),
            pl.BlockSpec((H, 1), lambda i: (0, 0)),
        ],
        out_specs=[
            pl.BlockSpec((br, H), lambda i: (i, 0)),
            pl.BlockSpec((br, 1), lambda i: (i, 0)),
            pl.BlockSpec((br, 1), lambda i: (i, 0)),
            pl.BlockSpec((br, 1), lambda i: (i, 0)),
        ],
        out_shape=[
            jax.ShapeDtypeStruct((NP, H), F32),
            jax.ShapeDtypeStruct((NP, 1), F32),
            jax.ShapeDtypeStruct((NP, 1), F32),
            jax.ShapeDtypeStruct((NP, 1), F32),
        ],
    )(xp, W_gat, asrc, adst)


def _tc_mlp_body(nump_ref, denp_ref, h_ref, es_ref, bgat_ref, w1_ref, b1_ref,
                 w2_ref, v_ref):
    es = es_ref[...]
    num = nump_ref[0] + nump_ref[1] + es * h_ref[...]
    den = denp_ref[0] + denp_ref[1] + es + 1e-16
    gat = num / den + bgat_ref[...]
    hp = jnp.maximum(gat, 0.0)
    mlp = jnp.dot(hp, w1_ref[...], preferred_element_type=F32) + b1_ref[...]
    mlp = jnp.maximum(mlp, 0.0)
    v_ref[...] = jnp.dot(mlp, w2_ref[...], preferred_element_type=F32)


def _tc_mlp(nump, denp, h, es, bgat, W1, b1, W2cols):
    br = 1264
    grid = NP // br
    return pl.pallas_call(
        _tc_mlp_body,
        grid=(grid,),
        in_specs=[
            pl.BlockSpec((2, br, H), lambda i: (0, i, 0)),
            pl.BlockSpec((2, br, 1), lambda i: (0, i, 0)),
            pl.BlockSpec((br, H), lambda i: (i, 0)),
            pl.BlockSpec((br, 1), lambda i: (i, 0)),
            pl.BlockSpec((1, H), lambda i: (0, 0)),
            pl.BlockSpec((H, H), lambda i: (0, 0)),
            pl.BlockSpec((1, H), lambda i: (0, 0)),
            pl.BlockSpec((H, 8), lambda i: (0, 0)),
        ],
        out_specs=pl.BlockSpec((br, 8), lambda i: (i, 0)),
        out_shape=jax.ShapeDtypeStruct((NP, 8), F32),
    )(nump, denp, h, es, bgat, W1, b1, W2cols)


def _tc_comb1_body(agg_ref, v_ref, deg_ref, w3_ref):
    agg = agg_ref[0] + agg_ref[1]
    v = v_ref[...]
    y = 0.5 * (deg_ref[...] * v + agg)
    row = lax.broadcasted_iota(I32, y.shape, 0)
    w3_ref[...] = jnp.where(row == 0, v - y, y)


def _tc_comb1(aggA, v3, deg):
    return pl.pallas_call(
        _tc_comb1_body,
        grid=(1,),
        in_specs=[
            pl.BlockSpec((2, 3, NP), lambda i: (0, 0, 0)),
            pl.BlockSpec((3, NP), lambda i: (0, 0)),
            pl.BlockSpec((1, NP), lambda i: (0, 0)),
        ],
        out_specs=pl.BlockSpec((3, NP), lambda i: (0, 0)),
        out_shape=jax.ShapeDtypeStruct((3, NP), F32),
    )(aggA, v3, deg)


def _tc_comb2_body(agg_ref, w3_ref, deg_ref, b2_ref, out_ref):
    agg = agg_ref[0] + agg_ref[1]
    w3 = w3_ref[...]
    z = 0.5 * (deg_ref[...] * w3 + agg)
    row = lax.broadcasted_iota(I32, w3.shape, 0)
    cw = jnp.where(row == 2, 0.0, 1.0)
    cz = jnp.where(row == 2, 1.0, -1.0)
    acc = jnp.sum(cw * w3 + cz * z, axis=0, keepdims=True)
    out_ref[...] = 0.5 * acc + b2_ref[...]


def _tc_comb2(aggB, w3, deg, b2):
    return pl.pallas_call(
        _tc_comb2_body,
        grid=(1,),
        in_specs=[
            pl.BlockSpec((2, 3, NP), lambda i: (0, 0, 0)),
            pl.BlockSpec((3, NP), lambda i: (0, 0)),
            pl.BlockSpec((1, NP), lambda i: (0, 0)),
            pl.BlockSpec((1, 1), lambda i: (0, 0)),
        ],
        out_specs=pl.BlockSpec((1, NP), lambda i: (0, 0)),
        out_shape=jax.ShapeDtypeStruct((1, NP), F32),
    )(aggB, w3, deg, b2)


@functools.cache
def _mesh():
    return plsc.VectorSubcoreMesh(core_axis_name="c", subcore_axis_name="s",
                                  num_cores=NC, num_subcores=NS)


def _gat_edges_body(nblk, src_hbm, dst_hbm, as_hbm, ad_hbm, h_hbm,
                    num_out, den_out, asb, adb, srcb, dstb, eeb, rows, sem,
                    num_sh, den_sh):
    c = lax.axis_index("c")
    s = lax.axis_index("s")
    wid = s * NC + c

    pltpu.sync_copy(as_hbm, asb)
    pltpu.sync_copy(ad_hbm, adb)

    def _zrow(i, _):
        for g in range(8):
            rows[i, pl.ds(g * 16, 16)] = jnp.zeros((16,), F32)
        return 0
    lax.fori_loop(0, BLK, _zrow, 0)
    for g in range(8):
        eeb[pl.ds(g * 16, 16)] = jnp.zeros((16,), F32)
    for k in range(Z_STRIPE // BLK):
        pltpu.sync_copy(rows, num_sh.at[pl.ds(s * Z_STRIPE + k * BLK, BLK)])
        pltpu.sync_copy(eeb, den_sh.at[pl.ds(s * Z_STRIPE + k * BLK, BLK)])
    plsc.subcore_barrier()

    base_e = wid * (nblk * BLK)

    def _block(b, _):
        off = base_e + b * BLK
        pltpu.sync_copy(src_hbm.at[pl.ds(off, BLK)], srcb)
        pltpu.sync_copy(dst_hbm.at[pl.ds(off, BLK)], dstb)
        for g in range(BLK // 16):
            s16 = srcb[pl.ds(g * 16, 16)]
            d16 = dstb[pl.ds(g * 16, 16)]
            e = plsc.load_gather(asb, [s16]) + plsc.load_gather(adb, [d16])
            e = jnp.where(e > 0.0, e, 0.2 * e)
            eeb[pl.ds(g * 16, 16)] = jnp.exp(e)
        pltpu.async_copy(h_hbm.at[srcb], rows, sem).wait()

        def _scale(r, _):
            spl = plsc.load_gather(eeb, [jnp.full((16,), r, I32)])
            for g in range(8):
                rows[r, pl.ds(g * 16, 16)] = rows[r, pl.ds(g * 16, 16)] * spl
            return 0
        lax.fori_loop(0, BLK, _scale, 0)

        pltpu.sync_copy(rows, num_sh.at[dstb], add=True)
        pltpu.sync_copy(eeb, den_sh.at[dstb], add=True)
        return 0
    lax.fori_loop(0, nblk, _block, 0)
    plsc.subcore_barrier()

    row0 = s * OUT_STRIPE
    for k in range(_cdiv(OUT_STRIPE, BLK)):
        cn = min(BLK, OUT_STRIPE - k * BLK)
        pltpu.sync_copy(num_sh.at[pl.ds(row0 + k * BLK, cn)],
                        rows.at[pl.ds(0, cn)])
        pltpu.sync_copy(rows.at[pl.ds(0, cn)],
                        num_out.at[pl.ds(c * NP + row0 + k * BLK, cn)])
        pltpu.sync_copy(den_sh.at[pl.ds(row0 + k * BLK, cn)],
                        eeb.at[pl.ds(0, cn)])
        pltpu.sync_copy(eeb.at[pl.ds(0, cn)],
                        den_out.at[pl.ds(c * NP + row0 + k * BLK, cn)])


def _gat_edges(src_p, dst_p, a_s, a_d, h, nblk):
    fn = pl.kernel(
        functools.partial(_gat_edges_body, nblk),
        out_type=[
            jax.ShapeDtypeStruct((NC * NP, H), F32),
            jax.ShapeDtypeStruct((NC * NP,), F32),
        ],
        mesh=_mesh(),
        compiler_params=pltpu.CompilerParams(needs_layout_passes=False),
        scratch_types=[
            pltpu.VMEM((NP,), F32),
            pltpu.VMEM((NP,), F32),
            pltpu.VMEM((BLK,), I32),
            pltpu.VMEM((BLK,), I32),
            pltpu.VMEM((BLK,), F32),
            pltpu.VMEM((BLK, H), F32),
            pltpu.SemaphoreType.DMA,
            pltpu.VMEM_SHARED((NACC, H), F32),
            pltpu.VMEM_SHARED((NACC,), F32),
        ],
    )
    return fn(src_p, dst_p, a_s, a_d, h)


def _spmv3_body(nblk, src_hbm, dst_hbm, lap_hbm, x_hbm, agg_out,
                x0b, x1b, x2b, srcb, dstb, lapb, v0b, v1b, v2b,
                acc0, acc1, acc2):
    c = lax.axis_index("c")
    s = lax.axis_index("s")
    wid = s * NC + c

    pltpu.sync_copy(x_hbm.at[pl.ds(0, NP)], x0b)
    pltpu.sync_copy(x_hbm.at[pl.ds(NP, NP)], x1b)
    pltpu.sync_copy(x_hbm.at[pl.ds(2 * NP, NP)], x2b)

    for g in range(8):
        v0b[pl.ds(g * 16, 16)] = jnp.zeros((16,), F32)
    for k in range(Z_STRIPE // BLK):
        sl = pl.ds(s * Z_STRIPE + k * BLK, BLK)
        pltpu.sync_copy(v0b, acc0.at[sl])
        pltpu.sync_copy(v0b, acc1.at[sl])
        pltpu.sync_copy(v0b, acc2.at[sl])
    plsc.subcore_barrier()

    base_e = wid * (nblk * BLK)

    def _block(b, _):
        off = base_e + b * BLK
        pltpu.sync_copy(src_hbm.at[pl.ds(off, BLK)], srcb)
        pltpu.sync_copy(dst_hbm.at[pl.ds(off, BLK)], dstb)
        pltpu.sync_copy(lap_hbm.at[pl.ds(off, BLK)], lapb)
        for g in range(BLK // 16):
            sl = pl.ds(g * 16, 16)
            s16 = srcb[sl]
            lp = lapb[sl]
            v0b[sl] = lp * plsc.load_gather(x0b, [s16])
            v1b[sl] = lp * plsc.load_gather(x1b, [s16])
            v2b[sl] = lp * plsc.load_gather(x2b, [s16])
        pltpu.sync_copy(v0b, acc0.at[dstb], add=True)
        pltpu.sync_copy(v1b, acc1.at[dstb], add=True)
        pltpu.sync_copy(v2b, acc2.at[dstb], add=True)
        return 0
    lax.fori_loop(0, nblk, _block, 0)
    plsc.subcore_barrier()

    row0 = s * OUT_STRIPE
    for k, acc in enumerate((acc0, acc1, acc2)):
        pltpu.sync_copy(acc.at[pl.ds(row0, OUT_STRIPE)],
                        x0b.at[pl.ds(0, OUT_STRIPE)])
        pltpu.sync_copy(
            x0b.at[pl.ds(0, OUT_STRIPE)],
            agg_out.at[pl.ds((c * 3 + k) * NP + row0, OUT_STRIPE)])


def _spmv3(src_p, dst_p, lap_p, x3, nblk):
    fn = pl.kernel(
        functools.partial(_spmv3_body, nblk),
        out_type=jax.ShapeDtypeStruct((NC * 3 * NP,), F32),
        mesh=_mesh(),
        compiler_params=pltpu.CompilerParams(needs_layout_passes=False),
        scratch_types=[
            pltpu.VMEM((NP,), F32),
            pltpu.VMEM((NP,), F32),
            pltpu.VMEM((NP,), F32),
            pltpu.VMEM((BLK,), I32),
            pltpu.VMEM((BLK,), I32),
            pltpu.VMEM((BLK,), F32),
            pltpu.VMEM((BLK,), F32),
            pltpu.VMEM((BLK,), F32),
            pltpu.VMEM((BLK,), F32),
            pltpu.VMEM_SHARED((NACC,), F32),
            pltpu.VMEM_SHARED((NACC,), F32),
            pltpu.VMEM_SHARED((NACC,), F32),
        ],
    )
    return fn(src_p, dst_p, lap_p, x3)


def kernel(x, edge_index, lap_values, deg_values, W_gat, att_src, att_dst,
           b_gat, W1, b1, W2, b2):
    E = edge_index.shape[1]
    nblk = _cdiv(E, NW * BLK)
    e_pad = NW * nblk * BLK

    src = edge_index[0]
    dst = edge_index[1]
    pad = e_pad - E
    src_p = jnp.concatenate([src, jnp.zeros((pad,), I32)])
    dst_p = jnp.concatenate([dst, jnp.full((pad,), DEAD, I32)])
    lap_p = jnp.concatenate([lap_values, jnp.zeros((pad,), F32)])

    xp = jnp.pad(x, ((0, NP - N), (0, 0)))
    degp = jnp.pad(deg_values, (0, NP - N)).reshape(1, NP)

    h, a_s, a_d, es = _tc_pre(xp, W_gat, att_src.reshape(H, 1),
                              att_dst.reshape(H, 1))

    nump, denp = _gat_edges(src_p, dst_p, a_s.reshape(NP), a_d.reshape(NP),
                            h, nblk)

    w2c = jnp.concatenate(
        [W2[:, 0].reshape(3, H).T, jnp.zeros((H, 5), F32)], axis=1)
    v8 = _tc_mlp(nump.reshape(NC, NP, H), denp.reshape(NC, NP, 1), h, es,
                 b_gat.reshape(1, H), W1, b1.reshape(1, H), w2c)
    v3 = v8[:, :3].T

    aggA = _spmv3(src_p, dst_p, lap_p, v3.reshape(3 * NP), nblk).reshape(
        NC, 3, NP)
    w3 = _tc_comb1(aggA, v3, degp)
    aggB = _spmv3(src_p, dst_p, lap_p, w3.reshape(3 * NP), nblk).reshape(
        NC, 3, NP)
    out = _tc_comb2(aggB, w3, degp, b2.reshape(1, 1))
    return out[0, :N]


# speedup vs baseline: 25.9662x; 1.5054x over previous
import functools

import jax
import jax.numpy as jnp
from jax import lax
from jax.experimental import pallas as pl
from jax.experimental.pallas import tpu as pltpu
from jax.experimental.pallas import tpu_sc as plsc

F32 = jnp.float32
I32 = jnp.int32

N = 10000
D = 128
H = 128
NP = 10112
NACC = 10240
DEAD = NP
NC, NS = 2, 16
NW = NC * NS
BLK = 128
OUT_STRIPE = 632
Z_STRIPE = 640
NRING = 4
CH = 40
CW = H // NC


def _cdiv(a, b):
    return (a + b - 1) // b


def _tc_pre_body(x_ref, wg_ref, asrc_ref, adst_ref, h_ref, as_ref, ad_ref,
                 es_ref):
    h = jnp.dot(x_ref[...], wg_ref[...], preferred_element_type=F32)
    h_ref[...] = h
    a_s = jnp.dot(h, asrc_ref[...], preferred_element_type=F32)
    a_d = jnp.dot(h, adst_ref[...], preferred_element_type=F32)
    as_ref[...] = a_s
    ad_ref[...] = a_d
    e = a_s + a_d
    e = jnp.where(e > 0.0, e, 0.2 * e)
    es_ref[...] = jnp.exp(e)


def _tc_pre(xp, W_gat, asrc, adst):
    br = 1264
    grid = NP // br
    return pl.pallas_call(
        _tc_pre_body,
        grid=(grid,),
        in_specs=[
            pl.BlockSpec((br, D), lambda i: (i, 0)),
            pl.BlockSpec((D, H), lambda i: (0, 0)),
            pl.BlockSpec((H, 1), lambda i: (0, 0)),
            pl.BlockSpec((H, 1), lambda i: (0, 0)),
        ],
        out_specs=[
            pl.BlockSpec((br, H), lambda i: (i, 0)),
            pl.BlockSpec((br, 1), lambda i: (i, 0)),
            pl.BlockSpec((br, 1), lambda i: (i, 0)),
            pl.BlockSpec((br, 1), lambda i: (i, 0)),
        ],
        out_shape=[
            jax.ShapeDtypeStruct((NP, H), F32),
            jax.ShapeDtypeStruct((NP, 1), F32),
            jax.ShapeDtypeStruct((NP, 1), F32),
            jax.ShapeDtypeStruct((NP, 1), F32),
        ],
    )(xp, W_gat, asrc, adst)


def _tc_mlp_body(nump_ref, denp_ref, h_ref, es_ref, bgat_ref, w1_ref, b1_ref,
                 w2_ref, v_ref):
    es = es_ref[...]
    num = jnp.concatenate([nump_ref[0], nump_ref[1]], axis=-1)
    num = num + es * h_ref[...]
    den = denp_ref[...] + es + 1e-16
    gat = num / den + bgat_ref[...]
    hp = jnp.maximum(gat, 0.0)
    mlp = jnp.dot(hp, w1_ref[...], preferred_element_type=F32) + b1_ref[...]
    mlp = jnp.maximum(mlp, 0.0)
    v_ref[...] = jnp.dot(mlp, w2_ref[...], preferred_element_type=F32)


def _tc_mlp(nump, denp, h, es, bgat, W1, b1, W2cols):
    br = 1264
    grid = NP // br
    return pl.pallas_call(
        _tc_mlp_body,
        grid=(grid,),
        in_specs=[
            pl.BlockSpec((2, br, CW), lambda i: (0, i, 0)),
            pl.BlockSpec((br, 1), lambda i: (i, 0)),
            pl.BlockSpec((br, H), lambda i: (i, 0)),
            pl.BlockSpec((br, 1), lambda i: (i, 0)),
            pl.BlockSpec((1, H), lambda i: (0, 0)),
            pl.BlockSpec((H, H), lambda i: (0, 0)),
            pl.BlockSpec((1, H), lambda i: (0, 0)),
            pl.BlockSpec((H, 8), lambda i: (0, 0)),
        ],
        out_specs=pl.BlockSpec((br, 8), lambda i: (i, 0)),
        out_shape=jax.ShapeDtypeStruct((NP, 8), F32),
    )(nump, denp, h, es, bgat, W1, b1, W2cols)


def _tc_comb1_body(agg_ref, v_ref, deg_ref, w3_ref):
    agg = agg_ref[0] + agg_ref[1]
    v = v_ref[...]
    y = 0.5 * (deg_ref[...] * v + agg)
    row = lax.broadcasted_iota(I32, y.shape, 0)
    w3_ref[...] = jnp.where(row == 0, v - y, y)


def _tc_comb1(aggA, v3, deg):
    return pl.pallas_call(
        _tc_comb1_body,
        grid=(1,),
        in_specs=[
            pl.BlockSpec((2, 3, NP), lambda i: (0, 0, 0)),
            pl.BlockSpec((3, NP), lambda i: (0, 0)),
            pl.BlockSpec((1, NP), lambda i: (0, 0)),
        ],
        out_specs=pl.BlockSpec((3, NP), lambda i: (0, 0)),
        out_shape=jax.ShapeDtypeStruct((3, NP), F32),
    )(aggA, v3, deg)


def _tc_comb2_body(agg_ref, w3_ref, deg_ref, b2_ref, out_ref):
    agg = agg_ref[0] + agg_ref[1]
    w3 = w3_ref[...]
    z = 0.5 * (deg_ref[...] * w3 + agg)
    row = lax.broadcasted_iota(I32, w3.shape, 0)
    cw = jnp.where(row == 2, 0.0, 1.0)
    cz = jnp.where(row == 2, 1.0, -1.0)
    acc = jnp.sum(cw * w3 + cz * z, axis=0, keepdims=True)
    out_ref[...] = 0.5 * acc + b2_ref[...]


def _tc_comb2(aggB, w3, deg, b2):
    return pl.pallas_call(
        _tc_comb2_body,
        grid=(1,),
        in_specs=[
            pl.BlockSpec((2, 3, NP), lambda i: (0, 0, 0)),
            pl.BlockSpec((3, NP), lambda i: (0, 0)),
            pl.BlockSpec((1, NP), lambda i: (0, 0)),
            pl.BlockSpec((1, 1), lambda i: (0, 0)),
        ],
        out_specs=pl.BlockSpec((1, NP), lambda i: (0, 0)),
        out_shape=jax.ShapeDtypeStruct((1, NP), F32),
    )(aggB, w3, deg, b2)


@functools.cache
def _mesh():
    return plsc.VectorSubcoreMesh(core_axis_name="c", subcore_axis_name="s",
                                  num_cores=NC, num_subcores=NS)


def _gat_edges_body(M, src_hbm, dst_hbm, as_hbm, ad_hbm, h2_hbm,
                    num_out, den_out, asb, adb, src2d, dst2d, ee2d, hix2d,
                    rows, gs0, gs1, gs2, gs3, ss0, ss1, ss2, ss3,
                    num_sh, den_sh):
    c = lax.axis_index("c")
    s = lax.axis_index("s")
    gsem = (gs0, gs1, gs2, gs3)
    ssem = (ss0, ss1, ss2, ss3)

    pltpu.sync_copy(as_hbm, asb)
    pltpu.sync_copy(ad_hbm, adb)

    def _zrow(i, _):
        for g in range(CW // 16):
            rows[0, i, pl.ds(g * 16, 16)] = jnp.zeros((16,), F32)
        return 0
    lax.fori_loop(0, BLK, _zrow, 0)
    for g in range(BLK // 16):
        ee2d[0, pl.ds(g * 16, 16)] = jnp.zeros((16,), F32)
    for k in range(Z_STRIPE // BLK):
        pltpu.sync_copy(rows.at[0],
                        num_sh.at[pl.ds(s * Z_STRIPE + k * BLK, BLK)])
        pltpu.sync_copy(ee2d.at[0],
                        den_sh.at[pl.ds(s * Z_STRIPE + k * BLK, BLK)])
    plsc.subcore_barrier()

    hbase = c * NP

    def _fire_gather(jl, t):
        pltpu.async_copy(h2_hbm.at[hix2d.at[jl]], rows.at[t], gsem[t])

    def _drain(t, sem):
        pltpu.make_async_copy(h2_hbm.at[pl.ds(0, BLK)], rows.at[t],
                              sem).wait()

    def _chunk(q, _):
        blk0 = s * M + q * CH
        pltpu.sync_copy(src_hbm.at[pl.ds(blk0, CH)], src2d)
        pltpu.sync_copy(dst_hbm.at[pl.ds(blk0, CH)], dst2d)

        def _ee(j, _):
            for g in range(BLK // 16):
                s16 = src2d[j, pl.ds(g * 16, 16)]
                d16 = dst2d[j, pl.ds(g * 16, 16)]
                e = plsc.load_gather(asb, [s16]) + plsc.load_gather(adb,
                                                                   [d16])
                e = jnp.where(e > 0.0, e, 0.2 * e)
                ee2d[j, pl.ds(g * 16, 16)] = jnp.exp(e)
                hix2d[j, pl.ds(g * 16, 16)] = s16 + hbase
            return 0
        lax.fori_loop(0, CH, _ee, 0)

        _fire_gather(0, 0)
        _fire_gather(1, 1)

        def _super(k, _):
            for t in range(NRING):
                jl = k * NRING + t
                t2 = (t + 2) % NRING

                @pl.when(jl >= 2)
                def _():
                    _drain(t2, ssem[t2])

                @pl.when(jl + 2 < CH)
                def _():
                    _fire_gather(jl + 2, t2)

                _drain(t, gsem[t])

                def _scale(r, _):
                    spl = plsc.load_gather(
                        ee2d,
                        [jnp.full((16,), jl, I32), jnp.full((16,), r, I32)])
                    for g in range(CW // 16):
                        rows[t, r, pl.ds(g * 16, 16)] = (
                            rows[t, r, pl.ds(g * 16, 16)] * spl)
                    return 0
                lax.fori_loop(0, BLK, _scale, 0)

                pltpu.async_copy(rows.at[t], num_sh.at[dst2d.at[jl]],
                                 ssem[t], add=True)

                @pl.when(c == 0)
                def _():
                    pltpu.sync_copy(ee2d.at[jl], den_sh.at[dst2d.at[jl]],
                                    add=True)
            return 0
        lax.fori_loop(0, CH // NRING, _super, 0)
        _drain((CH - 2) % NRING, ssem[(CH - 2) % NRING])
        _drain((CH - 1) % NRING, ssem[(CH - 1) % NRING])
        return 0
    lax.fori_loop(0, M // CH, _chunk, 0)
    plsc.subcore_barrier()

    row0 = s * OUT_STRIPE
    for k in range(_cdiv(OUT_STRIPE, BLK)):
        cn = min(BLK, OUT_STRIPE - k * BLK)
        pltpu.sync_copy(num_sh.at[pl.ds(row0 + k * BLK, cn)],
                        rows.at[0, pl.ds(0, cn)])
        pltpu.sync_copy(rows.at[0, pl.ds(0, cn)],
                        num_out.at[pl.ds(c * NP + row0 + k * BLK, cn)])

        @pl.when(c == 0)
        def _():
            pltpu.sync_copy(den_sh.at[pl.ds(row0 + k * BLK, cn)],
                            ee2d.at[0, pl.ds(0, cn)])
            pltpu.sync_copy(ee2d.at[0, pl.ds(0, cn)],
                            den_out.at[pl.ds(row0 + k * BLK, cn)])


def _gat_edges(src2, dst2, a_s, a_d, h2, M):
    fn = pl.kernel(
        functools.partial(_gat_edges_body, M),
        out_type=[
            jax.ShapeDtypeStruct((NC * NP, CW), F32),
            jax.ShapeDtypeStruct((NP,), F32),
        ],
        mesh=_mesh(),
        compiler_params=pltpu.CompilerParams(needs_layout_passes=False,
                                             use_tc_tiling_on_sc=False),
        scratch_types=[
            pltpu.VMEM((NP,), F32),
            pltpu.VMEM((NP,), F32),
            pltpu.VMEM((CH, BLK), I32),
            pltpu.VMEM((CH, BLK), I32),
            pltpu.VMEM((CH, BLK), F32),
            pltpu.VMEM((CH, BLK), I32),
            pltpu.VMEM((NRING, BLK, CW), F32),
        ] + [pltpu.SemaphoreType.DMA] * (2 * NRING) + [
            pltpu.VMEM_SHARED((NACC, CW), F32),
            pltpu.VMEM_SHARED((NACC,), F32),
        ],
    )
    return fn(src2, dst2, a_s, a_d, h2)


SPMV_Q = 8


def _spmv3_body(nblk, src_hbm, dst_hbm, lap_hbm, x_hbm, agg_out,
                x0b, x1b, x2b, src2d, dst2d, lap2d, v0, v1, v2, sem,
                acc0, acc1, acc2):
    c = lax.axis_index("c")
    s = lax.axis_index("s")
    wid = s * NC + c

    pltpu.sync_copy(x_hbm.at[pl.ds(0, NP)], x0b)
    pltpu.sync_copy(x_hbm.at[pl.ds(NP, NP)], x1b)
    pltpu.sync_copy(x_hbm.at[pl.ds(2 * NP, NP)], x2b)
    pltpu.sync_copy(src_hbm.at[pl.ds(wid * nblk, nblk)], src2d)
    pltpu.sync_copy(dst_hbm.at[pl.ds(wid * nblk, nblk)], dst2d)
    pltpu.sync_copy(lap_hbm.at[pl.ds(wid * nblk, nblk)], lap2d)

    for g in range(8):
        v0[0, pl.ds(g * 16, 16)] = jnp.zeros((16,), F32)
    for k in range(Z_STRIPE // BLK):
        sl = pl.ds(s * Z_STRIPE + k * BLK, BLK)
        pltpu.sync_copy(v0.at[0], acc0.at[sl])
        pltpu.sync_copy(v0.at[0], acc1.at[sl])
        pltpu.sync_copy(v0.at[0], acc2.at[sl])

    def _vals(j, _):
        for g in range(BLK // 16):
            sl = pl.ds(g * 16, 16)
            s16 = src2d[j, sl]
            lp = lap2d[j, sl]
            v0[j, sl] = lp * plsc.load_gather(x0b, [s16])
            v1[j, sl] = lp * plsc.load_gather(x1b, [s16])
            v2[j, sl] = lp * plsc.load_gather(x2b, [s16])
        return 0
    lax.fori_loop(0, nblk, _vals, 0)
    plsc.subcore_barrier()

    def _drain3(_i, _):
        for r in (v0, v1, v2):
            pltpu.make_async_copy(x_hbm.at[pl.ds(0, BLK)], r.at[0],
                                  sem).wait()
        return 0

    def _fire(j, _):
        pltpu.async_copy(v0.at[j], acc0.at[dst2d.at[j]], sem, add=True)
        pltpu.async_copy(v1.at[j], acc1.at[dst2d.at[j]], sem, add=True)
        pltpu.async_copy(v2.at[j], acc2.at[dst2d.at[j]], sem, add=True)

        @pl.when(j >= SPMV_Q)
        def _():
            _drain3(0, 0)
        return 0
    lax.fori_loop(0, nblk, _fire, 0)
    lax.fori_loop(0, SPMV_Q, _drain3, 0)
    plsc.subcore_barrier()

    row0 = s * OUT_STRIPE
    for k, acc in enumerate((acc0, acc1, acc2)):
        pltpu.sync_copy(acc.at[pl.ds(row0, OUT_STRIPE)],
                        x0b.at[pl.ds(0, OUT_STRIPE)])
        pltpu.sync_copy(
            x0b.at[pl.ds(0, OUT_STRIPE)],
            agg_out.at[pl.ds((c * 3 + k) * NP + row0, OUT_STRIPE)])


def _spmv3(src2, dst2, lap2, x3, nblk):
    fn = pl.kernel(
        functools.partial(_spmv3_body, nblk),
        out_type=jax.ShapeDtypeStruct((NC * 3 * NP,), F32),
        mesh=_mesh(),
        compiler_params=pltpu.CompilerParams(needs_layout_passes=False),
        scratch_types=[
            pltpu.VMEM((NP,), F32),
            pltpu.VMEM((NP,), F32),
            pltpu.VMEM((NP,), F32),
            pltpu.VMEM((nblk, BLK), I32),
            pltpu.VMEM((nblk, BLK), I32),
            pltpu.VMEM((nblk, BLK), F32),
            pltpu.VMEM((nblk, BLK), F32),
            pltpu.VMEM((nblk, BLK), F32),
            pltpu.VMEM((nblk, BLK), F32),
            pltpu.SemaphoreType.DMA,
            pltpu.VMEM_SHARED((NACC,), F32),
            pltpu.VMEM_SHARED((NACC,), F32),
            pltpu.VMEM_SHARED((NACC,), F32),
        ],
    )
    return fn(src2, dst2, lap2, x3)


def kernel(x, edge_index, lap_values, deg_values, W_gat, att_src, att_dst,
           b_gat, W1, b1, W2, b2):
    E = edge_index.shape[1]
    M = _cdiv(E, NS * BLK)
    M = _cdiv(M, CH) * CH
    e_pad = NS * M * BLK
    nblk = e_pad // (NW * BLK)

    src = edge_index[0]
    dst = edge_index[1]
    pad = e_pad - E
    src_p = jnp.concatenate([src, jnp.zeros((pad,), I32)]).reshape(-1, BLK)
    dst_p = jnp.concatenate([dst, jnp.full((pad,), DEAD, I32)]).reshape(
        -1, BLK)
    lap_p = jnp.concatenate([lap_values, jnp.zeros((pad,), F32)]).reshape(
        -1, BLK)

    xp = jnp.pad(x, ((0, NP - N), (0, 0)))
    degp = jnp.pad(deg_values, (0, NP - N)).reshape(1, NP)

    h, a_s, a_d, es = _tc_pre(xp, W_gat, att_src.reshape(H, 1),
                              att_dst.reshape(H, 1))

    h2 = h.reshape(NP, NC, CW).transpose(1, 0, 2).reshape(NC * NP, CW)
    nump, denp = _gat_edges(src_p, dst_p, a_s.reshape(NP), a_d.reshape(NP),
                            h2, M)

    w2c = jnp.concatenate(
        [W2[:, 0].reshape(3, H).T, jnp.zeros((H, 5), F32)], axis=1)
    v8 = _tc_mlp(nump.reshape(NC, NP, CW), denp.reshape(NP, 1), h, es,
                 b_gat.reshape(1, H), W1, b1.reshape(1, H), w2c)
    v3 = v8[:, :3].T

    aggA = _spmv3(src_p, dst_p, lap_p, v3.reshape(3 * NP), nblk).reshape(
        NC, 3, NP)
    w3 = _tc_comb1(aggA, v3, degp)
    aggB = _spmv3(src_p, dst_p, lap_p, w3.reshape(3 * NP), nblk).reshape(
        NC, 3, NP)
    out = _tc_comb2(aggB, w3, degp, b2.reshape(1, 1))
    return out[0, :N]


# speedup vs baseline: 26.1784x; 1.0082x over previous
import functools

import jax
import jax.numpy as jnp
from jax import lax
from jax.experimental import pallas as pl
from jax.experimental.pallas import tpu as pltpu
from jax.experimental.pallas import tpu_sc as plsc

F32 = jnp.float32
I32 = jnp.int32

N = 10000
D = 128
H = 128
NP = 10112
NACC = 10240
DEAD = NP
NC, NS = 2, 16
NW = NC * NS
BLK = 128
OUT_STRIPE = 632
Z_STRIPE = 640
NRING = 4
CH = 40
CW = H // NC


def _cdiv(a, b):
    return (a + b - 1) // b


def _tc_pre_body(x_ref, wg_ref, asrc_ref, adst_ref, h_ref, as_ref, ad_ref,
                 es_ref):
    h = jnp.dot(x_ref[...], wg_ref[...], preferred_element_type=F32)
    h_ref[...] = h
    a_s = jnp.dot(h, asrc_ref[...], preferred_element_type=F32)
    a_d = jnp.dot(h, adst_ref[...], preferred_element_type=F32)
    as_ref[...] = a_s
    ad_ref[...] = a_d
    e = a_s + a_d
    e = jnp.where(e > 0.0, e, 0.2 * e)
    es_ref[...] = jnp.exp(e)


def _tc_pre(xp, W_gat, asrc, adst):
    br = 1264
    grid = NP // br
    return pl.pallas_call(
        _tc_pre_body,
        grid=(grid,),
        in_specs=[
            pl.BlockSpec((br, D), lambda i: (i, 0)),
            pl.BlockSpec((D, H), lambda i: (0, 0)),
            pl.BlockSpec((H, 1), lambda i: (0, 0)),
            pl.BlockSpec((H, 1), lambda i: (0, 0)),
        ],
        out_specs=[
            pl.BlockSpec((br, H), lambda i: (i, 0)),
            pl.BlockSpec((br, 1), lambda i: (i, 0)),
            pl.BlockSpec((br, 1), lambda i: (i, 0)),
            pl.BlockSpec((br, 1), lambda i: (i, 0)),
        ],
        out_shape=[
            jax.ShapeDtypeStruct((NP, H), F32),
            jax.ShapeDtypeStruct((NP, 1), F32),
            jax.ShapeDtypeStruct((NP, 1), F32),
            jax.ShapeDtypeStruct((NP, 1), F32),
        ],
    )(xp, W_gat, asrc, adst)


def _tc_mlp_body(nump_ref, denp_ref, h_ref, es_ref, bgat_ref, w1_ref, b1_ref,
                 w2_ref, v_ref):
    es = es_ref[...]
    num = jnp.concatenate([nump_ref[0], nump_ref[1]], axis=-1)
    num = num + es * h_ref[...]
    den = denp_ref[...] + es + 1e-16
    gat = num / den + bgat_ref[...]
    hp = jnp.maximum(gat, 0.0)
    mlp = jnp.dot(hp, w1_ref[...], preferred_element_type=F32) + b1_ref[...]
    mlp = jnp.maximum(mlp, 0.0)
    v_ref[...] = jnp.dot(mlp, w2_ref[...], preferred_element_type=F32)


def _tc_mlp(nump, denp, h, es, bgat, W1, b1, W2cols):
    br = 1264
    grid = NP // br
    return pl.pallas_call(
        _tc_mlp_body,
        grid=(grid,),
        in_specs=[
            pl.BlockSpec((2, br, CW), lambda i: (0, i, 0)),
            pl.BlockSpec((br, 1), lambda i: (i, 0)),
            pl.BlockSpec((br, H), lambda i: (i, 0)),
            pl.BlockSpec((br, 1), lambda i: (i, 0)),
            pl.BlockSpec((1, H), lambda i: (0, 0)),
            pl.BlockSpec((H, H), lambda i: (0, 0)),
            pl.BlockSpec((1, H), lambda i: (0, 0)),
            pl.BlockSpec((H, 8), lambda i: (0, 0)),
        ],
        out_specs=pl.BlockSpec((br, 8), lambda i: (i, 0)),
        out_shape=jax.ShapeDtypeStruct((NP, 8), F32),
    )(nump, denp, h, es, bgat, W1, b1, W2cols)


def _tc_comb1_body(agg_ref, v_ref, deg_ref, w3_ref):
    agg = agg_ref[0] + agg_ref[1]
    v = v_ref[...]
    y = 0.5 * (deg_ref[...] * v + agg)
    row = lax.broadcasted_iota(I32, y.shape, 0)
    w3_ref[...] = jnp.where(row == 0, v - y, y)


def _tc_comb1(aggA, v3, deg):
    return pl.pallas_call(
        _tc_comb1_body,
        grid=(1,),
        in_specs=[
            pl.BlockSpec((2, 3, NP), lambda i: (0, 0, 0)),
            pl.BlockSpec((3, NP), lambda i: (0, 0)),
            pl.BlockSpec((1, NP), lambda i: (0, 0)),
        ],
        out_specs=pl.BlockSpec((3, NP), lambda i: (0, 0)),
        out_shape=jax.ShapeDtypeStruct((3, NP), F32),
    )(aggA, v3, deg)


def _tc_comb2_body(agg_ref, w3_ref, deg_ref, b2_ref, out_ref):
    agg = agg_ref[0] + agg_ref[1]
    w3 = w3_ref[...]
    z = 0.5 * (deg_ref[...] * w3 + agg)
    row = lax.broadcasted_iota(I32, w3.shape, 0)
    cw = jnp.where(row == 2, 0.0, 1.0)
    cz = jnp.where(row == 2, 1.0, -1.0)
    acc = jnp.sum(cw * w3 + cz * z, axis=0, keepdims=True)
    out_ref[...] = 0.5 * acc + b2_ref[...]


def _tc_comb2(aggB, w3, deg, b2):
    return pl.pallas_call(
        _tc_comb2_body,
        grid=(1,),
        in_specs=[
            pl.BlockSpec((2, 3, NP), lambda i: (0, 0, 0)),
            pl.BlockSpec((3, NP), lambda i: (0, 0)),
            pl.BlockSpec((1, NP), lambda i: (0, 0)),
            pl.BlockSpec((1, 1), lambda i: (0, 0)),
        ],
        out_specs=pl.BlockSpec((1, NP), lambda i: (0, 0)),
        out_shape=jax.ShapeDtypeStruct((1, NP), F32),
    )(aggB, w3, deg, b2)


@functools.cache
def _mesh():
    return plsc.VectorSubcoreMesh(core_axis_name="c", subcore_axis_name="s",
                                  num_cores=NC, num_subcores=NS)


def _gat_edges_body(M, src_hbm, dst_hbm, as_hbm, ad_hbm, h2_hbm,
                    num_out, den_out, asb, adb, src2d, dst2d, ee2d, hix2d,
                    rows, gs0, gs1, gs2, gs3, ss0, ss1, ss2, ss3, dsem,
                    num_sh, den_sh):
    c = lax.axis_index("c")
    s = lax.axis_index("s")
    gsem = (gs0, gs1, gs2, gs3)
    ssem = (ss0, ss1, ss2, ss3)

    pltpu.sync_copy(as_hbm, asb)
    pltpu.sync_copy(ad_hbm, adb)

    def _zrow(i, _):
        for g in range(CW // 16):
            rows[0, i, pl.ds(g * 16, 16)] = jnp.zeros((16,), F32)
        return 0
    lax.fori_loop(0, BLK, _zrow, 0)
    for g in range(BLK // 16):
        ee2d[0, pl.ds(g * 16, 16)] = jnp.zeros((16,), F32)
    for k in range(Z_STRIPE // BLK):
        pltpu.sync_copy(rows.at[0],
                        num_sh.at[pl.ds(s * Z_STRIPE + k * BLK, BLK)])
        pltpu.sync_copy(ee2d.at[0],
                        den_sh.at[pl.ds(s * Z_STRIPE + k * BLK, BLK)])
    plsc.subcore_barrier()

    hbase = c * NP

    def _fire_gather(jl, t):
        pltpu.async_copy(h2_hbm.at[hix2d.at[jl]], rows.at[t], gsem[t])

    def _drain(t, sem):
        pltpu.make_async_copy(h2_hbm.at[pl.ds(0, BLK)], rows.at[t],
                              sem).wait()

    def _chunk(q, _):
        blk0 = s * M + q * CH
        pltpu.sync_copy(src_hbm.at[pl.ds(blk0, CH)], src2d)
        pltpu.sync_copy(dst_hbm.at[pl.ds(blk0, CH)], dst2d)

        def _ee(j, _):
            for g in range(BLK // 16):
                s16 = src2d[j, pl.ds(g * 16, 16)]
                d16 = dst2d[j, pl.ds(g * 16, 16)]
                e = plsc.load_gather(asb, [s16]) + plsc.load_gather(adb,
                                                                   [d16])
                e = jnp.where(e > 0.0, e, 0.2 * e)
                ee2d[j, pl.ds(g * 16, 16)] = jnp.exp(e)
                hix2d[j, pl.ds(g * 16, 16)] = s16 + hbase
            return 0
        lax.fori_loop(0, CH, _ee, 0)

        _fire_gather(0, 0)
        _fire_gather(1, 1)

        def _super(k, _):
            for t in range(NRING):
                jl = k * NRING + t
                t2 = (t + 2) % NRING

                @pl.when(jl >= 2)
                def _():
                    _drain(t2, ssem[t2])

                @pl.when(jl + 2 < CH)
                def _():
                    _fire_gather(jl + 2, t2)

                _drain(t, gsem[t])

                jlv = jnp.full((16,), jl, I32)

                def _scale(r2, _):
                    for u in range(2):
                        r = r2 * 2 + u
                        spl = plsc.load_gather(
                            ee2d, [jlv, jnp.full((16,), r, I32)])
                        for g in range(CW // 16):
                            rows[t, r, pl.ds(g * 16, 16)] = (
                                rows[t, r, pl.ds(g * 16, 16)] * spl)
                    return 0
                lax.fori_loop(0, BLK // 2, _scale, 0)

                pltpu.async_copy(rows.at[t], num_sh.at[dst2d.at[jl]],
                                 ssem[t], add=True)

                @pl.when(c == 0)
                def _():
                    pltpu.async_copy(ee2d.at[jl], den_sh.at[dst2d.at[jl]],
                                     dsem, add=True)
            return 0
        lax.fori_loop(0, CH // NRING, _super, 0)
        _drain((CH - 2) % NRING, ssem[(CH - 2) % NRING])
        _drain((CH - 1) % NRING, ssem[(CH - 1) % NRING])

        @pl.when(c == 0)
        def _():
            def _dd(_i, _x):
                pltpu.make_async_copy(as_hbm.at[pl.ds(0, BLK)], ee2d.at[0],
                                      dsem).wait()
                return 0
            lax.fori_loop(0, CH, _dd, 0)
        return 0
    lax.fori_loop(0, M // CH, _chunk, 0)
    plsc.subcore_barrier()

    row0 = s * OUT_STRIPE
    for k in range(_cdiv(OUT_STRIPE, BLK)):
        cn = min(BLK, OUT_STRIPE - k * BLK)
        pltpu.sync_copy(num_sh.at[pl.ds(row0 + k * BLK, cn)],
                        rows.at[0, pl.ds(0, cn)])
        pltpu.sync_copy(rows.at[0, pl.ds(0, cn)],
                        num_out.at[pl.ds(c * NP + row0 + k * BLK, cn)])

        @pl.when(c == 0)
        def _():
            pltpu.sync_copy(den_sh.at[pl.ds(row0 + k * BLK, cn)],
                            ee2d.at[0, pl.ds(0, cn)])
            pltpu.sync_copy(ee2d.at[0, pl.ds(0, cn)],
                            den_out.at[pl.ds(row0 + k * BLK, cn)])


def _gat_edges(src2, dst2, a_s, a_d, h2, M):
    fn = pl.kernel(
        functools.partial(_gat_edges_body, M),
        out_type=[
            jax.ShapeDtypeStruct((NC * NP, CW), F32),
            jax.ShapeDtypeStruct((NP,), F32),
        ],
        mesh=_mesh(),
        compiler_params=pltpu.CompilerParams(needs_layout_passes=False,
                                             use_tc_tiling_on_sc=False),
        scratch_types=[
            pltpu.VMEM((NP,), F32),
            pltpu.VMEM((NP,), F32),
            pltpu.VMEM((CH, BLK), I32),
            pltpu.VMEM((CH, BLK), I32),
            pltpu.VMEM((CH, BLK), F32),
            pltpu.VMEM((CH, BLK), I32),
            pltpu.VMEM((NRING, BLK, CW), F32),
        ] + [pltpu.SemaphoreType.DMA] * (2 * NRING + 1) + [
            pltpu.VMEM_SHARED((NACC, CW), F32),
            pltpu.VMEM_SHARED((NACC,), F32),
        ],
    )
    return fn(src2, dst2, a_s, a_d, h2)


SPMV_Q = 8


def _spmv3_body(nblk, src_hbm, dst_hbm, lap_hbm, x_hbm, agg_out,
                x0b, x1b, x2b, src2d, dst2d, lap2d, v0, v1, v2, sem,
                acc0, acc1, acc2):
    c = lax.axis_index("c")
    s = lax.axis_index("s")
    wid = s * NC + c

    pltpu.sync_copy(x_hbm.at[pl.ds(0, NP)], x0b)
    pltpu.sync_copy(x_hbm.at[pl.ds(NP, NP)], x1b)
    pltpu.sync_copy(x_hbm.at[pl.ds(2 * NP, NP)], x2b)
    pltpu.sync_copy(src_hbm.at[pl.ds(wid * nblk, nblk)], src2d)
    pltpu.sync_copy(dst_hbm.at[pl.ds(wid * nblk, nblk)], dst2d)
    pltpu.sync_copy(lap_hbm.at[pl.ds(wid * nblk, nblk)], lap2d)

    for g in range(8):
        v0[0, pl.ds(g * 16, 16)] = jnp.zeros((16,), F32)
    for k in range(Z_STRIPE // BLK):
        sl = pl.ds(s * Z_STRIPE + k * BLK, BLK)
        pltpu.sync_copy(v0.at[0], acc0.at[sl])
        pltpu.sync_copy(v0.at[0], acc1.at[sl])
        pltpu.sync_copy(v0.at[0], acc2.at[sl])

    def _vals(j, _):
        for g in range(BLK // 16):
            sl = pl.ds(g * 16, 16)
            s16 = src2d[j, sl]
            lp = lap2d[j, sl]
            v0[j, sl] = lp * plsc.load_gather(x0b, [s16])
            v1[j, sl] = lp * plsc.load_gather(x1b, [s16])
            v2[j, sl] = lp * plsc.load_gather(x2b, [s16])
        return 0
    lax.fori_loop(0, nblk, _vals, 0)
    plsc.subcore_barrier()

    def _drain3(_i, _):
        for r in (v0, v1, v2):
            pltpu.make_async_copy(x_hbm.at[pl.ds(0, BLK)], r.at[0],
                                  sem).wait()
        return 0

    def _fire(j, _):
        pltpu.async_copy(v0.at[j], acc0.at[dst2d.at[j]], sem, add=True)
        pltpu.async_copy(v1.at[j], acc1.at[dst2d.at[j]], sem, add=True)
        pltpu.async_copy(v2.at[j], acc2.at[dst2d.at[j]], sem, add=True)

        @pl.when(j >= SPMV_Q)
        def _():
            _drain3(0, 0)
        return 0
    lax.fori_loop(0, nblk, _fire, 0)
    lax.fori_loop(0, SPMV_Q, _drain3, 0)
    plsc.subcore_barrier()

    row0 = s * OUT_STRIPE
    for k, acc in enumerate((acc0, acc1, acc2)):
        pltpu.sync_copy(acc.at[pl.ds(row0, OUT_STRIPE)],
                        x0b.at[pl.ds(0, OUT_STRIPE)])
        pltpu.sync_copy(
            x0b.at[pl.ds(0, OUT_STRIPE)],
            agg_out.at[pl.ds((c * 3 + k) * NP + row0, OUT_STRIPE)])


def _spmv3(src2, dst2, lap2, x3, nblk):
    fn = pl.kernel(
        functools.partial(_spmv3_body, nblk),
        out_type=jax.ShapeDtypeStruct((NC * 3 * NP,), F32),
        mesh=_mesh(),
        compiler_params=pltpu.CompilerParams(needs_layout_passes=False),
        scratch_types=[
            pltpu.VMEM((NP,), F32),
            pltpu.VMEM((NP,), F32),
            pltpu.VMEM((NP,), F32),
            pltpu.VMEM((nblk, BLK), I32),
            pltpu.VMEM((nblk, BLK), I32),
            pltpu.VMEM((nblk, BLK), F32),
            pltpu.VMEM((nblk, BLK), F32),
            pltpu.VMEM((nblk, BLK), F32),
            pltpu.VMEM((nblk, BLK), F32),
            pltpu.SemaphoreType.DMA,
            pltpu.VMEM_SHARED((NACC,), F32),
            pltpu.VMEM_SHARED((NACC,), F32),
            pltpu.VMEM_SHARED((NACC,), F32),
        ],
    )
    return fn(src2, dst2, lap2, x3)


def kernel(x, edge_index, lap_values, deg_values, W_gat, att_src, att_dst,
           b_gat, W1, b1, W2, b2):
    E = edge_index.shape[1]
    M = _cdiv(E, NS * BLK)
    M = _cdiv(M, CH) * CH
    e_pad = NS * M * BLK
    nblk = e_pad // (NW * BLK)

    src = edge_index[0]
    dst = edge_index[1]
    pad = e_pad - E
    src_p = jnp.concatenate([src, jnp.zeros((pad,), I32)]).reshape(-1, BLK)
    dst_p = jnp.concatenate([dst, jnp.full((pad,), DEAD, I32)]).reshape(
        -1, BLK)
    lap_p = jnp.concatenate([lap_values, jnp.zeros((pad,), F32)]).reshape(
        -1, BLK)

    xp = jnp.pad(x, ((0, NP - N), (0, 0)))
    degp = jnp.pad(deg_values, (0, NP - N)).reshape(1, NP)

    h, a_s, a_d, es = _tc_pre(xp, W_gat, att_src.reshape(H, 1),
                              att_dst.reshape(H, 1))

    h2 = h.reshape(NP, NC, CW).transpose(1, 0, 2).reshape(NC * NP, CW)
    nump, denp = _gat_edges(src_p, dst_p, a_s.reshape(NP), a_d.reshape(NP),
                            h2, M)

    w2c = jnp.concatenate(
        [W2[:, 0].reshape(3, H).T, jnp.zeros((H, 5), F32)], axis=1)
    v8 = _tc_mlp(nump.reshape(NC, NP, CW), denp.reshape(NP, 1), h, es,
                 b_gat.reshape(1, H), W1, b1.reshape(1, H), w2c)
    v3 = v8[:, :3].T

    aggA = _spmv3(src_p, dst_p, lap_p, v3.reshape(3 * NP), nblk).reshape(
        NC, 3, NP)
    w3 = _tc_comb1(aggA, v3, degp)
    aggB = _spmv3(src_p, dst_p, lap_p, w3.reshape(3 * NP), nblk).reshape(
        NC, 3, NP)
    out = _tc_comb2(aggB, w3, degp, b2.reshape(1, 1))
    return out[0, :N]


# speedup vs baseline: 28.6765x; 1.0954x over previous
import functools

import jax
import jax.numpy as jnp
from jax import lax
from jax.experimental import pallas as pl
from jax.experimental.pallas import tpu as pltpu
from jax.experimental.pallas import tpu_sc as plsc

F32 = jnp.float32
I32 = jnp.int32

N = 10000
D = 128
H = 128
NP = 10112
NACC = 10240
DEAD = NP
NC, NS = 2, 16
NW = NC * NS
BLK = 128
OUT_STRIPE = 632
Z_STRIPE = 640
NRING = 4
CH = 40
CW = H // NC


def _cdiv(a, b):
    return (a + b - 1) // b


def _tc_pre_body(x_ref, wg_ref, asrc_ref, adst_ref, h_ref, as_ref, ad_ref,
                 es_ref):
    h = jnp.dot(x_ref[...], wg_ref[...], preferred_element_type=F32)
    h_ref[...] = h
    a_s = jnp.dot(h, asrc_ref[...], preferred_element_type=F32)
    a_d = jnp.dot(h, adst_ref[...], preferred_element_type=F32)
    as_ref[...] = a_s
    ad_ref[...] = a_d
    e = a_s + a_d
    e = jnp.where(e > 0.0, e, 0.2 * e)
    es_ref[...] = jnp.exp(e)


def _tc_pre(xp, W_gat, asrc, adst):
    br = 1264
    grid = NP // br
    return pl.pallas_call(
        _tc_pre_body,
        grid=(grid,),
        in_specs=[
            pl.BlockSpec((br, D), lambda i: (i, 0)),
            pl.BlockSpec((D, H), lambda i: (0, 0)),
            pl.BlockSpec((H, 1), lambda i: (0, 0)),
            pl.BlockSpec((H, 1), lambda i: (0, 0)),
        ],
        out_specs=[
            pl.BlockSpec((br, H), lambda i: (i, 0)),
            pl.BlockSpec((br, 1), lambda i: (i, 0)),
            pl.BlockSpec((br, 1), lambda i: (i, 0)),
            pl.BlockSpec((br, 1), lambda i: (i, 0)),
        ],
        out_shape=[
            jax.ShapeDtypeStruct((NP, H), F32),
            jax.ShapeDtypeStruct((NP, 1), F32),
            jax.ShapeDtypeStruct((NP, 1), F32),
            jax.ShapeDtypeStruct((NP, 1), F32),
        ],
    )(xp, W_gat, asrc, adst)


def _tc_mlp_body(nump_ref, denp_ref, h_ref, es_ref, bgat_ref, w1_ref, b1_ref,
                 w2_ref, v_ref):
    es = es_ref[...]
    num = jnp.concatenate([nump_ref[0], nump_ref[1]], axis=-1)
    num = num + es * h_ref[...]
    den = denp_ref[...] + es + 1e-16
    gat = num / den + bgat_ref[...]
    hp = jnp.maximum(gat, 0.0)
    mlp = jnp.dot(hp, w1_ref[...], preferred_element_type=F32) + b1_ref[...]
    mlp = jnp.maximum(mlp, 0.0)
    v_ref[...] = jnp.dot(mlp, w2_ref[...], preferred_element_type=F32)


def _tc_mlp(nump, denp, h, es, bgat, W1, b1, W2cols):
    br = 1264
    grid = NP // br
    return pl.pallas_call(
        _tc_mlp_body,
        grid=(grid,),
        in_specs=[
            pl.BlockSpec((2, br, CW), lambda i: (0, i, 0)),
            pl.BlockSpec((br, 1), lambda i: (i, 0)),
            pl.BlockSpec((br, H), lambda i: (i, 0)),
            pl.BlockSpec((br, 1), lambda i: (i, 0)),
            pl.BlockSpec((1, H), lambda i: (0, 0)),
            pl.BlockSpec((H, H), lambda i: (0, 0)),
            pl.BlockSpec((1, H), lambda i: (0, 0)),
            pl.BlockSpec((H, 8), lambda i: (0, 0)),
        ],
        out_specs=pl.BlockSpec((br, 8), lambda i: (i, 0)),
        out_shape=jax.ShapeDtypeStruct((NP, 8), F32),
    )(nump, denp, h, es, bgat, W1, b1, W2cols)


def _tc_comb1_body(agg_ref, v_ref, deg_ref, w3_ref):
    agg = agg_ref[0] + agg_ref[1]
    v = v_ref[...]
    y = 0.5 * (deg_ref[...] * v + agg)
    row = lax.broadcasted_iota(I32, y.shape, 0)
    w3_ref[...] = jnp.where(row == 0, v - y, y)


def _tc_comb1(aggA, v3, deg):
    return pl.pallas_call(
        _tc_comb1_body,
        grid=(1,),
        in_specs=[
            pl.BlockSpec((2, 3, NP), lambda i: (0, 0, 0)),
            pl.BlockSpec((3, NP), lambda i: (0, 0)),
            pl.BlockSpec((1, NP), lambda i: (0, 0)),
        ],
        out_specs=pl.BlockSpec((3, NP), lambda i: (0, 0)),
        out_shape=jax.ShapeDtypeStruct((3, NP), F32),
    )(aggA, v3, deg)


def _tc_comb2_body(agg_ref, w3_ref, deg_ref, b2_ref, out_ref):
    agg = agg_ref[0] + agg_ref[1]
    w3 = w3_ref[...]
    z = 0.5 * (deg_ref[...] * w3 + agg)
    row = lax.broadcasted_iota(I32, w3.shape, 0)
    cw = jnp.where(row == 2, 0.0, 1.0)
    cz = jnp.where(row == 2, 1.0, -1.0)
    acc = jnp.sum(cw * w3 + cz * z, axis=0, keepdims=True)
    out_ref[...] = 0.5 * acc + b2_ref[...]


def _tc_comb2(aggB, w3, deg, b2):
    return pl.pallas_call(
        _tc_comb2_body,
        grid=(1,),
        in_specs=[
            pl.BlockSpec((2, 3, NP), lambda i: (0, 0, 0)),
            pl.BlockSpec((3, NP), lambda i: (0, 0)),
            pl.BlockSpec((1, NP), lambda i: (0, 0)),
            pl.BlockSpec((1, 1), lambda i: (0, 0)),
        ],
        out_specs=pl.BlockSpec((1, NP), lambda i: (0, 0)),
        out_shape=jax.ShapeDtypeStruct((1, NP), F32),
    )(aggB, w3, deg, b2)


@functools.cache
def _mesh():
    return plsc.VectorSubcoreMesh(core_axis_name="c", subcore_axis_name="s",
                                  num_cores=NC, num_subcores=NS)


def _gat_edges_body(M, src_hbm, dst_hbm, as_hbm, ad_hbm, h2_hbm,
                    num_out, den_out, asb, adb, src2d, dst2d, ee2d, hix2d,
                    rows, srows, gs0, gs1, gs2, gs3, ss0, ss1, dsem,
                    num_sh, den_sh):
    c = lax.axis_index("c")
    s = lax.axis_index("s")
    gsem = (gs0, gs1, gs2, gs3)
    ssem = (ss0, ss1)

    pltpu.sync_copy(as_hbm, asb)
    pltpu.sync_copy(ad_hbm, adb)

    def _zrow(i, _):
        for g in range(CW // 16):
            srows[0, i, pl.ds(g * 16, 16)] = jnp.zeros((16,), F32)
        return 0
    lax.fori_loop(0, BLK, _zrow, 0)
    for g in range(BLK // 16):
        ee2d[0, pl.ds(g * 16, 16)] = jnp.zeros((16,), F32)
    for k in range(Z_STRIPE // BLK):
        pltpu.sync_copy(srows.at[0],
                        num_sh.at[pl.ds(s * Z_STRIPE + k * BLK, BLK)])
        pltpu.sync_copy(ee2d.at[0],
                        den_sh.at[pl.ds(s * Z_STRIPE + k * BLK, BLK)])
    plsc.subcore_barrier()

    hbase = c * NP

    def _fire_gather(jl, t):
        pltpu.async_copy(h2_hbm.at[hix2d.at[jl]], rows.at[t], gsem[t])

    def _drain(t, sem):
        pltpu.make_async_copy(h2_hbm.at[pl.ds(0, BLK)], rows.at[t],
                              sem).wait()

    def _sdrain(sp, sem):
        pltpu.make_async_copy(num_out.at[pl.ds(0, BLK)], srows.at[sp],
                              sem).wait()

    def _chunk(q, _):
        blk0 = s * M + q * CH
        pltpu.sync_copy(src_hbm.at[pl.ds(blk0, CH)], src2d)
        pltpu.sync_copy(dst_hbm.at[pl.ds(blk0, CH)], dst2d)

        def _ee(j, _):
            for g in range(BLK // 16):
                s16 = src2d[j, pl.ds(g * 16, 16)]
                d16 = dst2d[j, pl.ds(g * 16, 16)]
                e = plsc.load_gather(asb, [s16]) + plsc.load_gather(adb,
                                                                   [d16])
                e = jnp.where(e > 0.0, e, 0.2 * e)
                ee2d[j, pl.ds(g * 16, 16)] = jnp.exp(e)
                hix2d[j, pl.ds(g * 16, 16)] = s16 + hbase
            return 0
        lax.fori_loop(0, CH, _ee, 0)

        _fire_gather(0, 0)
        _fire_gather(1, 1)

        def _super(k, _):
            for t in range(NRING):
                jl = k * NRING + t
                t2 = (t + 2) % NRING

                sp = t % 2

                @pl.when(jl >= 2)
                def _():
                    _sdrain(sp, ssem[sp])

                @pl.when(jl + 2 < CH)
                def _():
                    _fire_gather(jl + 2, t2)

                _drain(t, gsem[t])

                jlv = jnp.full((16,), jl, I32)

                def _scale(r2, _):
                    for u in range(2):
                        r = r2 * 2 + u
                        spl = plsc.load_gather(
                            ee2d, [jlv, jnp.full((16,), r, I32)])
                        for g in range(CW // 32):
                            x = rows[t, r, pl.ds(g * 32, 32)]
                            a, b = plsc.unpack(
                                x, format=plsc.PackFormat.INTERLEAVED)
                            srows[sp, r, pl.ds(g * 32, 16)] = a * spl
                            srows[sp, r, pl.ds(g * 32 + 16, 16)] = b * spl
                    return 0
                lax.fori_loop(0, BLK // 2, _scale, 0)

                pltpu.async_copy(srows.at[sp], num_sh.at[dst2d.at[jl]],
                                 ssem[sp], add=True)

                @pl.when(c == 0)
                def _():
                    pltpu.async_copy(ee2d.at[jl], den_sh.at[dst2d.at[jl]],
                                     dsem, add=True)
            return 0
        lax.fori_loop(0, CH // NRING, _super, 0)
        _sdrain(0, ssem[0])
        _sdrain(1, ssem[1])

        @pl.when(c == 0)
        def _():
            def _dd(_i, _x):
                pltpu.make_async_copy(as_hbm.at[pl.ds(0, BLK)], ee2d.at[0],
                                      dsem).wait()
                return 0
            lax.fori_loop(0, CH, _dd, 0)
        return 0
    lax.fori_loop(0, M // CH, _chunk, 0)
    plsc.subcore_barrier()

    row0 = s * OUT_STRIPE
    for k in range(_cdiv(OUT_STRIPE, BLK)):
        cn = min(BLK, OUT_STRIPE - k * BLK)
        pltpu.sync_copy(num_sh.at[pl.ds(row0 + k * BLK, cn)],
                        srows.at[0, pl.ds(0, cn)])
        pltpu.sync_copy(srows.at[0, pl.ds(0, cn)],
                        num_out.at[pl.ds(c * NP + row0 + k * BLK, cn)])

        @pl.when(c == 0)
        def _():
            pltpu.sync_copy(den_sh.at[pl.ds(row0 + k * BLK, cn)],
                            ee2d.at[0, pl.ds(0, cn)])
            pltpu.sync_copy(ee2d.at[0, pl.ds(0, cn)],
                            den_out.at[pl.ds(row0 + k * BLK, cn)])


def _gat_edges(src2, dst2, a_s, a_d, h2, M):
    fn = pl.kernel(
        functools.partial(_gat_edges_body, M),
        out_type=[
            jax.ShapeDtypeStruct((NC * NP, CW), F32),
            jax.ShapeDtypeStruct((NP,), F32),
        ],
        mesh=_mesh(),
        compiler_params=pltpu.CompilerParams(needs_layout_passes=False,
                                             use_tc_tiling_on_sc=False),
        scratch_types=[
            pltpu.VMEM((NP,), F32),
            pltpu.VMEM((NP,), F32),
            pltpu.VMEM((CH, BLK), I32),
            pltpu.VMEM((CH, BLK), I32),
            pltpu.VMEM((CH, BLK), F32),
            pltpu.VMEM((CH, BLK), I32),
            pltpu.VMEM((NRING, BLK, CW), jnp.bfloat16),
            pltpu.VMEM((2, BLK, CW), F32),
        ] + [pltpu.SemaphoreType.DMA] * (NRING + 3) + [
            pltpu.VMEM_SHARED((NACC, CW), F32),
            pltpu.VMEM_SHARED((NACC,), F32),
        ],
    )
    return fn(src2, dst2, a_s, a_d, h2)


SPMV_Q = 8


def _spmv3_body(nblk, src_hbm, dst_hbm, lap_hbm, x_hbm, agg_out,
                x0b, x1b, x2b, src2d, dst2d, lap2d, v0, v1, v2, sem,
                acc0, acc1, acc2):
    c = lax.axis_index("c")
    s = lax.axis_index("s")
    wid = s * NC + c

    pltpu.sync_copy(x_hbm.at[pl.ds(0, NP)], x0b)
    pltpu.sync_copy(x_hbm.at[pl.ds(NP, NP)], x1b)
    pltpu.sync_copy(x_hbm.at[pl.ds(2 * NP, NP)], x2b)
    pltpu.sync_copy(src_hbm.at[pl.ds(wid * nblk, nblk)], src2d)
    pltpu.sync_copy(dst_hbm.at[pl.ds(wid * nblk, nblk)], dst2d)
    pltpu.sync_copy(lap_hbm.at[pl.ds(wid * nblk, nblk)], lap2d)

    for g in range(8):
        v0[0, pl.ds(g * 16, 16)] = jnp.zeros((16,), F32)
    for k in range(Z_STRIPE // BLK):
        sl = pl.ds(s * Z_STRIPE + k * BLK, BLK)
        pltpu.sync_copy(v0.at[0], acc0.at[sl])
        pltpu.sync_copy(v0.at[0], acc1.at[sl])
        pltpu.sync_copy(v0.at[0], acc2.at[sl])

    def _vals(j, _):
        for g in range(BLK // 16):
            sl = pl.ds(g * 16, 16)
            s16 = src2d[j, sl]
            lp = lap2d[j, sl]
            v0[j, sl] = lp * plsc.load_gather(x0b, [s16])
            v1[j, sl] = lp * plsc.load_gather(x1b, [s16])
            v2[j, sl] = lp * plsc.load_gather(x2b, [s16])
        return 0
    lax.fori_loop(0, nblk, _vals, 0)
    plsc.subcore_barrier()

    def _drain3(_i, _):
        for r in (v0, v1, v2):
            pltpu.make_async_copy(x_hbm.at[pl.ds(0, BLK)], r.at[0],
                                  sem).wait()
        return 0

    def _fire(j, _):
        pltpu.async_copy(v0.at[j], acc0.at[dst2d.at[j]], sem, add=True)
        pltpu.async_copy(v1.at[j], acc1.at[dst2d.at[j]], sem, add=True)
        pltpu.async_copy(v2.at[j], acc2.at[dst2d.at[j]], sem, add=True)

        @pl.when(j >= SPMV_Q)
        def _():
            _drain3(0, 0)
        return 0
    lax.fori_loop(0, nblk, _fire, 0)
    lax.fori_loop(0, SPMV_Q, _drain3, 0)
    plsc.subcore_barrier()

    row0 = s * OUT_STRIPE
    for k, acc in enumerate((acc0, acc1, acc2)):
        pltpu.sync_copy(acc.at[pl.ds(row0, OUT_STRIPE)],
                        x0b.at[pl.ds(0, OUT_STRIPE)])
        pltpu.sync_copy(
            x0b.at[pl.ds(0, OUT_STRIPE)],
            agg_out.at[pl.ds((c * 3 + k) * NP + row0, OUT_STRIPE)])


def _spmv3(src2, dst2, lap2, x3, nblk):
    fn = pl.kernel(
        functools.partial(_spmv3_body, nblk),
        out_type=jax.ShapeDtypeStruct((NC * 3 * NP,), F32),
        mesh=_mesh(),
        compiler_params=pltpu.CompilerParams(needs_layout_passes=False),
        scratch_types=[
            pltpu.VMEM((NP,), F32),
            pltpu.VMEM((NP,), F32),
            pltpu.VMEM((NP,), F32),
            pltpu.VMEM((nblk, BLK), I32),
            pltpu.VMEM((nblk, BLK), I32),
            pltpu.VMEM((nblk, BLK), F32),
            pltpu.VMEM((nblk, BLK), F32),
            pltpu.VMEM((nblk, BLK), F32),
            pltpu.VMEM((nblk, BLK), F32),
            pltpu.SemaphoreType.DMA,
            pltpu.VMEM_SHARED((NACC,), F32),
            pltpu.VMEM_SHARED((NACC,), F32),
            pltpu.VMEM_SHARED((NACC,), F32),
        ],
    )
    return fn(src2, dst2, lap2, x3)


def kernel(x, edge_index, lap_values, deg_values, W_gat, att_src, att_dst,
           b_gat, W1, b1, W2, b2):
    E = edge_index.shape[1]
    M = _cdiv(E, NS * BLK)
    M = _cdiv(M, CH) * CH
    e_pad = NS * M * BLK
    nblk = e_pad // (NW * BLK)

    src = edge_index[0]
    dst = edge_index[1]
    pad = e_pad - E
    src_p = jnp.concatenate([src, jnp.zeros((pad,), I32)]).reshape(-1, BLK)
    dst_p = jnp.concatenate([dst, jnp.full((pad,), DEAD, I32)]).reshape(
        -1, BLK)
    lap_p = jnp.concatenate([lap_values, jnp.zeros((pad,), F32)]).reshape(
        -1, BLK)

    xp = jnp.pad(x, ((0, NP - N), (0, 0)))
    degp = jnp.pad(deg_values, (0, NP - N)).reshape(1, NP)

    h, a_s, a_d, es = _tc_pre(xp, W_gat, att_src.reshape(H, 1),
                              att_dst.reshape(H, 1))

    h2 = h.reshape(NP, NC, CW).transpose(1, 0, 2).reshape(NC * NP, CW)
    perm = []
    for g in range(CW // 32):
        for i in range(16):
            perm += [g * 32 + i, g * 32 + 16 + i]
    h2p = h2[:, jnp.array(perm, jnp.int32)]
    nump, denp = _gat_edges(src_p, dst_p, a_s.reshape(NP), a_d.reshape(NP),
                            h2p.astype(jnp.bfloat16), M)

    w2c = jnp.concatenate(
        [W2[:, 0].reshape(3, H).T, jnp.zeros((H, 5), F32)], axis=1)
    v8 = _tc_mlp(nump.reshape(NC, NP, CW), denp.reshape(NP, 1), h, es,
                 b_gat.reshape(1, H), W1, b1.reshape(1, H), w2c)
    v3 = v8[:, :3].T

    aggA = _spmv3(src_p, dst_p, lap_p, v3.reshape(3 * NP), nblk).reshape(
        NC, 3, NP)
    w3 = _tc_comb1(aggA, v3, degp)
    aggB = _spmv3(src_p, dst_p, lap_p, w3.reshape(3 * NP), nblk).reshape(
        NC, 3, NP)
    out = _tc_comb2(aggB, w3, degp, b2.reshape(1, 1))
    return out[0, :N]
